# baseline (speedup 1.0000x reference)
import jax
import jax.numpy as jnp
from jax import lax
from jax.experimental import pallas as pl
from jax.experimental.pallas import tpu as pltpu
from jax.experimental.pallas import tpu_sc as plsc

_VOCAB_SIZE = 1000
_TBL = 2048
_N = 4096 * 20
_NW = 32
_PER_W = _N // _NW
_L = 16
_FULL = _VOCAB_SIZE // _L


def _lookup_body(inp_hbm, vocab_hbm, out_hbm, inp_v, vocab_v, inv_v, out_v,
                 inp_sem, vocab_sem):
    wid = lax.axis_index("s") * 2 + lax.axis_index("c")
    base = wid * _PER_W
    inp_dma = pltpu.async_copy(
        inp_hbm.at[pl.ds(2 * base, 2 * _PER_W)], inp_v, inp_sem)
    vocab_dma = pltpu.async_copy(vocab_hbm, vocab_v, vocab_sem)

    lane = lax.iota(jnp.int32, _L)
    lane2 = lane * 2
    oov = jnp.full((_L,), _VOCAB_SIZE, jnp.int32)
    zeros = jnp.zeros((_L,), jnp.int32)

    def init_step(i, carry):
        inv_v[pl.ds(i * _L, _L)] = oov
        return carry

    lax.fori_loop(0, _TBL // _L, init_step, 0, unroll=8)

    def zero_step(i, carry):
        out_v[pl.ds(i * _L, _L)] = zeros
        return carry

    lax.fori_loop(0, 2 * _PER_W // _L, zero_step, 0, unroll=8)
    vocab_dma.wait()

    def scatter_step(j, carry):
        k_a = plsc.load_gather(vocab_v, [lane2 + 2 * (j * _L)])
        k_b = plsc.load_gather(vocab_v, [lane2 + 2 * (j * _L) + 1])
        keys = k_a | k_b
        plsc.store_scatter(inv_v, [keys], lane + j * _L)
        return carry

    lax.fori_loop(0, _FULL, scatter_step, 0, unroll=8)
    t_a = plsc.load_gather(vocab_v, [lane2 + 2 * (_VOCAB_SIZE - _L)])
    t_b = plsc.load_gather(vocab_v, [lane2 + 2 * (_VOCAB_SIZE - _L) + 1])
    tail_keys = t_a | t_b
    plsc.store_scatter(inv_v, [tail_keys], lane + (_VOCAB_SIZE - _L))

    inp_dma.wait()

    def gather_step(i, carry):
        off2 = 2 * (i * _L)
        w_a = plsc.load_gather(inp_v, [lane2 + off2])
        w_b = plsc.load_gather(inp_v, [lane2 + off2 + 1])
        toks = w_a | w_b
        res = plsc.load_gather(inv_v, [toks])
        plsc.store_scatter(out_v, [lane2 + off2], res)
        return carry

    lax.fori_loop(0, _PER_W // _L, gather_step, 0, unroll=8)

    pltpu.sync_copy(out_v, out_hbm.at[pl.ds(2 * base, 2 * _PER_W)])


@jax.jit
def _lookup(inp_words, vocab_words):
    with jax.enable_x64(False):
        mesh = plsc.VectorSubcoreMesh(core_axis_name="c", subcore_axis_name="s")
        run = pl.kernel(
            _lookup_body,
            out_type=jax.ShapeDtypeStruct((2 * _N,), jnp.int32),
            mesh=mesh,
            scratch_types=[
                pltpu.VMEM((2 * _PER_W,), jnp.int32),
                pltpu.VMEM((2 * _VOCAB_SIZE,), jnp.int32),
                pltpu.VMEM((_TBL,), jnp.int32),
                pltpu.VMEM((2 * _PER_W,), jnp.int32),
                pltpu.SemaphoreType.DMA,
                pltpu.SemaphoreType.DMA,
            ],
            compiler_params=pltpu.CompilerParams(needs_layout_passes=False),
        )
        return run(inp_words, vocab_words)


def kernel(inputs, vocabulary):
    inp_words = lax.bitcast_convert_type(inputs.T, jnp.int32).reshape(-1)
    vocab_words = lax.bitcast_convert_type(vocabulary, jnp.int32).reshape(-1)
    out_words = _lookup(inp_words, vocab_words)
    out64 = lax.bitcast_convert_type(
        out_words.reshape(inputs.shape[1], inputs.shape[0], 2), jnp.int64)
    return out64.T


# speedup vs baseline: 5.8784x; 5.8784x over previous
import jax
import jax.numpy as jnp
from jax import lax
from jax.experimental import pallas as pl
from jax.experimental.pallas import tpu as pltpu
from jax.experimental.pallas import tpu_sc as plsc

_VOCAB_SIZE = 1000
_TBL = 2048
_N = 4096 * 20
_NW = 32
_PER_W = _N // _NW
_L = 16
_FULL = _VOCAB_SIZE // _L


def _lookup_body(inp_hbm, vocab_hbm, out_hbm, inp_v, vocab_v, inv_v, out_v,
                 inp_sem, vocab_sem):
    wid = lax.axis_index("s") * 2 + lax.axis_index("c")
    base = wid * _PER_W
    inp_dma = pltpu.async_copy(inp_hbm.at[pl.ds(base, _PER_W)], inp_v, inp_sem)
    vocab_dma = pltpu.async_copy(vocab_hbm, vocab_v, vocab_sem)

    lane = lax.iota(jnp.int32, _L)
    oov = jnp.full((_L,), _VOCAB_SIZE, jnp.int32)

    def init_step(i, carry):
        inv_v[pl.ds(i * _L, _L)] = oov
        return carry

    lax.fori_loop(0, _TBL // _L, init_step, 0, unroll=2)
    vocab_dma.wait()

    def scatter_step(j, carry):
        keys = plsc.bitcast(vocab_v[pl.ds(j * _L, _L)], jnp.int32)
        plsc.store_scatter(inv_v, [keys], lane + j * _L)
        return carry

    lax.fori_loop(0, _FULL, scatter_step, 0, unroll=2)
    tail_keys = plsc.bitcast(vocab_v[pl.ds(_VOCAB_SIZE - _L, _L)], jnp.int32)
    plsc.store_scatter(inv_v, [tail_keys], lane + (_VOCAB_SIZE - _L))

    inp_dma.wait()

    def gather_step(i, carry):
        off = i * _L
        toks = plsc.bitcast(inp_v[pl.ds(off, _L)], jnp.int32)
        out_v[pl.ds(off, _L)] = plsc.bitcast(
            plsc.load_gather(inv_v, [toks]), jnp.uint32)
        return carry

    lax.fori_loop(0, _PER_W // _L, gather_step, 0, unroll=2)

    pltpu.sync_copy(out_v, out_hbm.at[pl.ds(base, _PER_W)])


@jax.jit
def _lookup(flat_inputs, vocab):
    with jax.enable_x64(False):
        mesh = plsc.VectorSubcoreMesh(core_axis_name="c", subcore_axis_name="s")
        run = pl.kernel(
            _lookup_body,
            out_type=jax.ShapeDtypeStruct((_N,), jnp.uint32),
            mesh=mesh,
            scratch_types=[
                pltpu.VMEM((_PER_W,), jnp.uint32),
                pltpu.VMEM((_VOCAB_SIZE,), jnp.uint32),
                pltpu.VMEM((_TBL,), jnp.int32),
                pltpu.VMEM((_PER_W,), jnp.uint32),
                pltpu.SemaphoreType.DMA,
                pltpu.SemaphoreType.DMA,
            ],
            compiler_params=pltpu.CompilerParams(needs_layout_passes=False),
        )
        return run(flat_inputs, vocab)


def kernel(inputs, vocabulary):
    flat = inputs.astype(jnp.uint32).T.reshape(-1)
    vocab = vocabulary.astype(jnp.uint32)
    out = _lookup(flat, vocab)
    return out.reshape(inputs.shape[::-1]).T.astype(jnp.int64)
